# baseline (speedup 1.0000x reference)
import functools

import jax
import jax.numpy as jnp
from jax import lax
from jax.experimental import pallas as pl
from jax.experimental.pallas import tpu as pltpu
from jax.experimental.pallas import tpu_sc as plsc

B = 1024
L = 200
D = 64
N = B * L
NC = 2
NS = 16
LANES = 16
NW = NC * NS
CHUNK = 128
NUNITS = N // CHUNK
PER_W = NUNITS // NW
NBUF = 5
NBLK = PER_W // NBUF
DEPTH = 4
GRP = CHUNK // LANES
DHI = D // 8

_mesh = plsc.VectorSubcoreMesh(
    core_axis_name="c", subcore_axis_name="s", num_cores=NC, num_subcores=NS
)


@functools.partial(
    pl.kernel,
    out_type=jax.ShapeDtypeStruct((L, DHI, B // CHUNK, 8, CHUNK), jnp.float32),
    mesh=_mesh,
    compiler_params=pltpu.CompilerParams(
        use_tc_tiling_on_sc=False, needs_layout_passes=False
    ),
    scratch_types=[
        pltpu.VMEM((PER_W, CHUNK), jnp.int32),
        pltpu.VMEM((PER_W, CHUNK), jnp.float32),
    ]
    + [pltpu.VMEM((CHUNK, D), jnp.float32) for _ in range(NBUF)]
    + [pltpu.VMEM((DHI, 8, CHUNK), jnp.float32) for _ in range(NBUF)]
    + [
        pltpu.SemaphoreType.DMA((NBUF,)),
        pltpu.SemaphoreType.DMA((NBUF,)),
    ],
)
def _sc_embed(table_hbm, idx_hbm, val_hbm, out_hbm, idx_v, val_v, *bufs):
    rin = bufs[:NBUF]
    rout = bufs[NBUF:2 * NBUF]
    gsem, wsem = bufs[2 * NBUF], bufs[2 * NBUF + 1]

    wid = lax.axis_index("s") * NC + lax.axis_index("c")
    ubase = wid * PER_W

    pltpu.sync_copy(idx_hbm.at[pl.ds(ubase, PER_W)], idx_v)
    pltpu.sync_copy(val_hbm.at[pl.ds(ubase, PER_W)], val_v)

    for b in range(DEPTH):
        pltpu.async_copy(
            table_hbm.at[idx_v.at[b]], rin[b], gsem.at[b]
        )

    def block(kk, carry):
        for b in range(NBUF):
            c = kk * NBUF + b
            u = ubase + c
            l = u // (B // CHUNK)
            kb = u % (B // CHUNK)
            pltpu.make_async_copy(
                table_hbm.at[idx_v.at[c]], rin[b], gsem.at[b]
            ).wait()
            @pl.when(kk > 0)
            def _():
                for dh in range(DHI):
                    pltpu.make_async_copy(
                        rout[b].at[dh], out_hbm.at[l, dh, kb], wsem.at[b]
                    ).wait()

            def grp_body(g, carry2):
                toks = g * LANES + lax.iota(jnp.int32, LANES)
                v16 = val_v[c, pl.ds(g * LANES, LANES)]

                @plsc.parallel_loop(0, D, 1, unroll=8)
                def _dloop(d):
                    dvec = jnp.full((LANES,), d, jnp.int32)
                    src = plsc.load_gather(rin[b], [toks, dvec])
                    rout[b][d // 8, d % 8, pl.ds(g * LANES, LANES)] = src * v16

                return carry2

            lax.fori_loop(0, GRP, grp_body, 0)

            for dh in range(DHI):
                pltpu.async_copy(
                    rout[b].at[dh], out_hbm.at[l, dh, kb], wsem.at[b]
                )
            cn = jnp.minimum(c + DEPTH, PER_W - 1)

            @pl.when(c + DEPTH < PER_W)
            def _():
                pltpu.async_copy(
                    table_hbm.at[idx_v.at[cn]],
                    rin[(b + DEPTH) % NBUF],
                    gsem.at[(b + DEPTH) % NBUF],
                )

        return carry

    lax.fori_loop(0, NBLK, block, 0)

    for b in range(NBUF):
        c = (NBLK - 1) * NBUF + b
        u = ubase + c
        l = u // (B // CHUNK)
        kb = u % (B // CHUNK)
        for dh in range(DHI):
            pltpu.make_async_copy(
                rout[b].at[dh], out_hbm.at[l, dh, kb], wsem.at[b]
            ).wait()


def kernel(feature_idx, feature_val, table):
    idx_t = feature_idx.T.reshape(NUNITS, CHUNK).astype(jnp.int32)
    val_t = feature_val.reshape(B, L).T.reshape(NUNITS, CHUNK)
    out5 = _sc_embed(table, idx_t, val_t)
    return out5.transpose(2, 4, 0, 1, 3).reshape(B, L, D)


# speedup vs baseline: 2.0320x; 2.0320x over previous
import functools

import jax
import jax.numpy as jnp
from jax import lax
from jax.experimental import pallas as pl
from jax.experimental.pallas import tpu as pltpu
from jax.experimental.pallas import tpu_sc as plsc

B = 1024
L = 200
D = 64
N = B * L
NC = 2
NS = 16
LANES = 16
NW = NC * NS
CHUNK = 128
NUNITS = N // CHUNK
PER_W = NUNITS // NW
NBUF = 5
NBLK = PER_W // NBUF
DEPTH = 4
GRP = CHUNK // LANES
DHI = D // 8

_mesh = plsc.VectorSubcoreMesh(
    core_axis_name="c", subcore_axis_name="s", num_cores=NC, num_subcores=NS
)


@functools.partial(
    pl.kernel,
    out_type=jax.ShapeDtypeStruct((L, DHI, B // CHUNK, 8, CHUNK), jnp.float32),
    mesh=_mesh,
    compiler_params=pltpu.CompilerParams(
        use_tc_tiling_on_sc=False, needs_layout_passes=False
    ),
    scratch_types=[
        pltpu.VMEM((PER_W, CHUNK), jnp.int32),
        pltpu.VMEM((PER_W, CHUNK), jnp.float32),
    ]
    + [pltpu.VMEM((CHUNK, D), jnp.float32) for _ in range(NBUF)]
    + [pltpu.VMEM((DHI, 8, CHUNK), jnp.float32) for _ in range(NBUF)]
    + [
        pltpu.SemaphoreType.DMA((NBUF,)),
        pltpu.SemaphoreType.DMA((NBUF,)),
    ],
)
def _sc_embed(table_hbm, idx_hbm, val_hbm, out_hbm, idx_v, val_v, *bufs):
    rin = bufs[:NBUF]
    rout = bufs[NBUF:2 * NBUF]
    gsem, wsem = bufs[2 * NBUF], bufs[2 * NBUF + 1]

    wid = lax.axis_index("s") * NC + lax.axis_index("c")
    ubase = wid * PER_W

    pltpu.sync_copy(idx_hbm.at[pl.ds(ubase, PER_W)], idx_v)
    pltpu.sync_copy(val_hbm.at[pl.ds(ubase, PER_W)], val_v)

    for b in range(DEPTH):
        pltpu.async_copy(
            table_hbm.at[idx_v.at[b]], rin[b], gsem.at[b]
        )

    def block(kk, carry):
        for b in range(NBUF):
            c = kk * NBUF + b
            u = ubase + c
            l = u // (B // CHUNK)
            kb = u % (B // CHUNK)
            pltpu.make_async_copy(
                table_hbm.at[idx_v.at[c]], rin[b], gsem.at[b]
            ).wait()
            @pl.when(kk > 0)
            def _():
                for dh in range(DHI):
                    pltpu.make_async_copy(
                        rout[b].at[dh], out_hbm.at[l, dh, kb], wsem.at[b]
                    ).wait()

            def grp_body(g, carry2):
                lane = lax.iota(jnp.int32, LANES)
                toks = g * LANES + lane
                v16 = val_v[c, pl.ds(g * LANES, LANES)]

                @plsc.parallel_loop(0, D, 1, unroll=8)
                def _dloop(dp):
                    dvec = (dp + lane) & (D - 1)
                    src = plsc.load_gather(rin[b], [toks, dvec])
                    plsc.store_scatter(
                        rout[b], [dvec >> 3, dvec & 7, toks], src * v16
                    )

                return carry2

            lax.fori_loop(0, GRP, grp_body, 0)

            for dh in range(DHI):
                pltpu.async_copy(
                    rout[b].at[dh], out_hbm.at[l, dh, kb], wsem.at[b]
                )
            cn = jnp.minimum(c + DEPTH, PER_W - 1)

            @pl.when(c + DEPTH < PER_W)
            def _():
                pltpu.async_copy(
                    table_hbm.at[idx_v.at[cn]],
                    rin[(b + DEPTH) % NBUF],
                    gsem.at[(b + DEPTH) % NBUF],
                )

        return carry

    lax.fori_loop(0, NBLK, block, 0)

    for b in range(NBUF):
        c = (NBLK - 1) * NBUF + b
        u = ubase + c
        l = u // (B // CHUNK)
        kb = u % (B // CHUNK)
        for dh in range(DHI):
            pltpu.make_async_copy(
                rout[b].at[dh], out_hbm.at[l, dh, kb], wsem.at[b]
            ).wait()


def kernel(feature_idx, feature_val, table):
    idx_t = feature_idx.T.reshape(NUNITS, CHUNK).astype(jnp.int32)
    val_t = feature_val.reshape(B, L).T.reshape(NUNITS, CHUNK)
    out5 = _sc_embed(table, idx_t, val_t)
    return out5.transpose(2, 4, 0, 1, 3).reshape(B, L, D)
